# Initial kernel scaffold; baseline (speedup 1.0000x reference)
import jax
import jax.numpy as jnp
from jax.experimental import pallas as pl


def kernel(anchors, box_preds, cls_preds, dir_cls_preds):
    raise NotImplementedError("write your pallas kernel here")


# speedup vs baseline: 627.2257x; 627.2257x over previous
import math

import jax
import jax.numpy as jnp
from jax.experimental import pallas as pl

_INTERPRET = False

SCORE_THR = 0.3
IOU_THR = 0.01
PRE = 1000
PREP = 1024
POST = 100
TWO_PI = 2.0 * math.pi
TI = 8
TJ = 128


def _mask_kernel(x_ref, o_ref):
    s = jax.nn.sigmoid(x_ref[...])
    o_ref[...] = jnp.where(s > SCORE_THR, s, -jnp.inf)


def _decode_kernel(cand_ref, sc_ref, outbox_ref, corn_ref):
    c = cand_ref[0]
    sc = sc_ref[0]
    col = lambda a, i: jax.lax.slice_in_dim(a, i, i + 1, axis=1)
    xt, yt, zt, wt, lt, ht, rt = (col(c, i) for i in range(7))
    xa, ya, za, wa, la, ha, ra = (col(c, 7 + i) for i in range(7))
    d0, d1 = col(c, 14), col(c, 15)
    za = za + ha * 0.5
    diag = jnp.sqrt(la * la + wa * wa)
    xg = xt * diag + xa
    yg = yt * diag + ya
    zg = zt * ha + za
    lg = jnp.exp(lt) * la
    wg = jnp.exp(wt) * wa
    hg = jnp.exp(ht) * ha
    rg = rt + ra
    zg = zg - hg * 0.5
    dirlab = (d1 > d0).astype(jnp.float32)
    ang = rg - jnp.floor(rg / TWO_PI + 1.0) * TWO_PI + TWO_PI * dirlab
    score = jnp.where(sc > SCORE_THR, sc, 0.0)
    outbox_ref[0] = jnp.concatenate([xg, yg, zg, wg, lg, hg, ang, score], axis=1)
    ca, sa = jnp.cos(rg), jnp.sin(rg)
    hw, hl = wg * 0.5, lg * 0.5
    dxs = (-hw, hw, hw, -hw)
    dys = (-hl, -hl, hl, hl)
    cols = [ca * dxs[k] - sa * dys[k] + xg for k in range(4)]
    cols += [sa * dxs[k] + ca * dys[k] + yg for k in range(4)]
    cols.append(jnp.abs(wg * lg))
    z = jnp.zeros_like(xg)
    cols += [z] * 7
    corn_ref[0] = jnp.concatenate(cols, axis=1)


def _overlap_kernel(ci_ref, cj_ref, ov_ref):
    ci = ci_ref[0]
    cj = cj_ref[0]
    colT = lambda a, i: jax.lax.slice_in_dim(a, i, i + 1, axis=1)
    row = lambda a, i: jax.lax.slice_in_dim(a, i, i + 1, axis=0)
    px = [jnp.broadcast_to(colT(ci, k), (TI, TJ)) for k in range(4)]
    py = [jnp.broadcast_to(colT(ci, 4 + k), (TI, TJ)) for k in range(4)]
    area_i = colT(ci, 8)
    cxj = [row(cj, k) for k in range(4)]
    cyj = [row(cj, 4 + k) for k in range(4)]
    area_j = row(cj, 8)
    cnt = jnp.full((TI, TJ), 4, jnp.int32)
    for e in range(4):
        ax, ay = cxj[e], cyj[e]
        ex = cxj[(e + 1) % 4] - ax
        ey = cyj[(e + 1) % 4] - ay
        s_in = 4 + e
        s_out = 5 + e
        cp = [ex * (py[s] - ay) - ey * (px[s] - ax) for s in range(s_in)]
        nx = [jnp.zeros((TI, TJ), jnp.float32) for _ in range(s_out)]
        ny = [jnp.zeros((TI, TJ), jnp.float32) for _ in range(s_out)]
        off = jnp.zeros((TI, TJ), jnp.int32)
        for s in range(s_in):
            vs = cnt > s
            if s == s_in - 1:
                qx, qy, cq = px[0], py[0], cp[0]
            else:
                wrap = cnt == (s + 1)
                qx = jnp.where(wrap, px[0], px[s + 1])
                qy = jnp.where(wrap, py[0], py[s + 1])
                cq = jnp.where(wrap, cp[0], cp[s + 1])
            ge_p = cp[s] >= 0
            ge_q = cq >= 0
            keep = vs & ge_p
            cross = vs & (ge_p != ge_q)
            denom = jnp.where(cross, cp[s] - cq, 1.0)
            t = cp[s] / denom
            ix = px[s] + t * (qx - px[s])
            iy = py[s] + t * (qy - py[s])
            for o in range(min(2 * s + 1, s_out)):
                c1 = keep & (off == o)
                nx[o] = jnp.where(c1, px[s], nx[o])
                ny[o] = jnp.where(c1, py[s], ny[o])
            off1 = off + keep.astype(jnp.int32)
            for o in range(min(2 * s + 2, s_out)):
                c2 = cross & (off1 == o)
                nx[o] = jnp.where(c2, ix, nx[o])
                ny[o] = jnp.where(c2, iy, ny[o])
            off = off1 + cross.astype(jnp.int32)
        px, py, cnt = nx, ny, off
    acc = jnp.zeros((TI, TJ), jnp.float32)
    for s in range(8):
        vs = cnt > s
        if s == 7:
            qx, qy = px[0], py[0]
        else:
            wrap = cnt == (s + 1)
            qx = jnp.where(wrap, px[0], px[s + 1])
            qy = jnp.where(wrap, py[0], py[s + 1])
        acc = acc + jnp.where(vs, px[s] * qy - qx * py[s], 0.0)
    inter = jnp.abs(acc) * 0.5
    union = area_i + area_j - inter
    ov = (union > 0) & (inter > IOU_THR * union)
    ov_ref[0] = ov.astype(jnp.float32)


def _nms_kernel(ov_ref, box_ref, sc_ref, sel_ref, cnt_ref):
    srow = sc_ref[0]
    lane = jax.lax.broadcasted_iota(jnp.int32, (1, PREP), 1)
    supp0 = (srow <= SCORE_THR).astype(jnp.float32)

    def body(i, supp):
        onehot = (lane == i).astype(jnp.float32)
        si = jnp.sum(supp * onehot)
        rowi = ov_ref[0, pl.ds(i, 1), :]
        gt = (lane > i).astype(jnp.float32)
        cand = jnp.maximum(supp, rowi * gt)
        return jnp.where(si == 0.0, cand, supp)

    supp = jax.lax.fori_loop(0, PRE, body, supp0)
    kept = 1.0 - supp
    li = jax.lax.broadcasted_iota(jnp.int32, (PREP, PREP), 0)
    lj = jax.lax.broadcasted_iota(jnp.int32, (PREP, PREP), 1)
    tri = (li <= lj).astype(jnp.float32)
    pos_incl = jnp.dot(kept, tri, preferred_element_type=jnp.float32)
    pos = (pos_incl - kept).astype(jnp.int32)
    rows = jax.lax.broadcasted_iota(jnp.int32, (128, PREP), 0)
    sel_m = ((rows == jnp.broadcast_to(pos, (128, PREP)))
             & (jnp.broadcast_to(kept, (128, PREP)) > 0)).astype(jnp.float32)
    sel = jnp.dot(sel_m, box_ref[0], preferred_element_type=jnp.float32)
    sel_ref[0] = sel
    k = jnp.minimum(jnp.sum(kept), float(POST))
    cnt_ref[0] = jnp.reshape(k.astype(jnp.int32), (1, 1))


def kernel(anchors, box_preds, cls_preds, dir_cls_preds):
    B, N = cls_preds.shape[0], cls_preds.shape[1]
    NP = 20480
    logits = cls_preds[..., 0]
    lp = jnp.pad(logits, ((0, 0), (0, NP - N)), constant_values=-1e9)
    masked = pl.pallas_call(
        _mask_kernel,
        out_shape=jax.ShapeDtypeStruct((B * NP // 128, 128), jnp.float32),
        interpret=_INTERPRET,
    )(lp.reshape(B * NP // 128, 128))
    masked = masked.reshape(B, NP)
    top_vals, top_idx = jax.lax.top_k(masked, PRE)
    cand = jnp.concatenate([box_preds, anchors, dir_cls_preds], axis=-1)
    g = jnp.take_along_axis(cand, top_idx[..., None], axis=1)
    g = jnp.pad(g, ((0, 0), (0, PREP - PRE), (0, 0)))
    sc = jnp.pad(top_vals, ((0, 0), (0, PREP - PRE)), constant_values=-jnp.inf)

    outbox, cornT = pl.pallas_call(
        _decode_kernel,
        grid=(B,),
        in_specs=[pl.BlockSpec((1, PREP, 16), lambda b: (b, 0, 0)),
                  pl.BlockSpec((1, PREP, 1), lambda b: (b, 0, 0))],
        out_specs=[pl.BlockSpec((1, PREP, 8), lambda b: (b, 0, 0)),
                   pl.BlockSpec((1, PREP, 16), lambda b: (b, 0, 0))],
        out_shape=[jax.ShapeDtypeStruct((B, PREP, 8), jnp.float32),
                   jax.ShapeDtypeStruct((B, PREP, 16), jnp.float32)],
        interpret=_INTERPRET,
    )(g, sc[..., None])
    corn_by = jnp.swapaxes(cornT, 1, 2)

    ovm = pl.pallas_call(
        _overlap_kernel,
        grid=(B, PREP // TI, PREP // TJ),
        in_specs=[pl.BlockSpec((1, TI, 16), lambda b, i, j: (b, i, 0)),
                  pl.BlockSpec((1, 16, TJ), lambda b, i, j: (b, 0, j))],
        out_specs=pl.BlockSpec((1, TI, TJ), lambda b, i, j: (b, i, j)),
        out_shape=jax.ShapeDtypeStruct((B, PREP, PREP), jnp.float32),
        interpret=_INTERPRET,
    )(cornT, corn_by)

    sel, cnt = pl.pallas_call(
        _nms_kernel,
        grid=(B,),
        in_specs=[pl.BlockSpec((1, PREP, PREP), lambda b: (b, 0, 0)),
                  pl.BlockSpec((1, PREP, 8), lambda b: (b, 0, 0)),
                  pl.BlockSpec((1, 1, PREP), lambda b: (b, 0, 0))],
        out_specs=[pl.BlockSpec((1, 128, 8), lambda b: (b, 0, 0)),
                   pl.BlockSpec((1, 1, 1), lambda b: (b, 0, 0))],
        out_shape=[jax.ShapeDtypeStruct((B, 128, 8), jnp.float32),
                   jax.ShapeDtypeStruct((B, 1, 1), jnp.int32)],
        interpret=_INTERPRET,
    )(ovm, outbox, sc[:, None, :])

    boxes = sel[:, :POST, :7]
    scores = sel[:, :POST, 7]
    labels = jnp.zeros((B, POST), jnp.int32)
    counts = cnt.reshape(B)
    return boxes, scores, labels, counts


# speedup vs baseline: 1033.6519x; 1.6480x over previous
import math

import jax
import jax.numpy as jnp
from jax.experimental import pallas as pl

_INTERPRET = False

SCORE_THR = 0.3
IOU_THR = 0.01
PRE = 1000
PREP = 1024
POST = 100
TWO_PI = 2.0 * math.pi
TI = 8
TJ = 128

_N_PAIR_TILES = 8 * (PREP // TJ) * (PREP // TJ + 1)


def _pair_tile_ij(t):
    tf = (2 * t + 4).astype(jnp.float32)
    j = jnp.floor((jnp.sqrt(tf) - 2.0) * 0.25).astype(jnp.int32)
    i = t - 8 * j * (j + 1)
    return i, j


def _mask_kernel(x_ref, o_ref):
    s = jax.nn.sigmoid(x_ref[...])
    o_ref[...] = jnp.where(s > SCORE_THR, s, -jnp.inf)


def _decode_kernel(cand_ref, sc_ref, outbox_ref, corn_ref):
    c = cand_ref[0]
    sc = sc_ref[0]
    col = lambda a, i: jax.lax.slice_in_dim(a, i, i + 1, axis=1)
    xt, yt, zt, wt, lt, ht, rt = (col(c, i) for i in range(7))
    xa, ya, za, wa, la, ha, ra = (col(c, 7 + i) for i in range(7))
    d0, d1 = col(c, 14), col(c, 15)
    za = za + ha * 0.5
    diag = jnp.sqrt(la * la + wa * wa)
    xg = xt * diag + xa
    yg = yt * diag + ya
    zg = zt * ha + za
    lg = jnp.exp(lt) * la
    wg = jnp.exp(wt) * wa
    hg = jnp.exp(ht) * ha
    rg = rt + ra
    zg = zg - hg * 0.5
    dirlab = (d1 > d0).astype(jnp.float32)
    ang = rg - jnp.floor(rg / TWO_PI + 1.0) * TWO_PI + TWO_PI * dirlab
    score = jnp.where(sc > SCORE_THR, sc, 0.0)
    outbox_ref[0] = jnp.concatenate([xg, yg, zg, wg, lg, hg, ang, score], axis=1)
    ca, sa = jnp.cos(rg), jnp.sin(rg)
    hw, hl = wg * 0.5, lg * 0.5
    dxs = (-hw, hw, hw, -hw)
    dys = (-hl, -hl, hl, hl)
    cols = [ca * dxs[k] - sa * dys[k] + xg for k in range(4)]
    cols += [sa * dxs[k] + ca * dys[k] + yg for k in range(4)]
    cols.append(jnp.abs(wg * lg))
    z = jnp.zeros_like(xg)
    cols += [z] * 7
    corn_ref[0] = jnp.concatenate(cols, axis=1)


def _overlap_kernel(ci_ref, cj_ref, ov_ref):
    ci = ci_ref[0]
    cj = cj_ref[0]
    colT = lambda a, i: jax.lax.slice_in_dim(a, i, i + 1, axis=1)
    row = lambda a, i: jax.lax.slice_in_dim(a, i, i + 1, axis=0)
    px = [jnp.broadcast_to(colT(ci, k), (TI, TJ)) for k in range(4)]
    py = [jnp.broadcast_to(colT(ci, 4 + k), (TI, TJ)) for k in range(4)]
    area_i = colT(ci, 8)
    cxj = [row(cj, k) for k in range(4)]
    cyj = [row(cj, 4 + k) for k in range(4)]
    area_j = row(cj, 8)
    cnt = jnp.full((TI, TJ), 4, jnp.int32)
    for e in range(4):
        ax, ay = cxj[e], cyj[e]
        ex = cxj[(e + 1) % 4] - ax
        ey = cyj[(e + 1) % 4] - ay
        s_in = 4 + e
        s_out = 5 + e
        cp = [ex * (py[s] - ay) - ey * (px[s] - ax) for s in range(s_in)]
        nx = [jnp.zeros((TI, TJ), jnp.float32) for _ in range(s_out)]
        ny = [jnp.zeros((TI, TJ), jnp.float32) for _ in range(s_out)]
        off = jnp.zeros((TI, TJ), jnp.int32)
        for s in range(s_in):
            vs = cnt > s
            if s == s_in - 1:
                qx, qy, cq = px[0], py[0], cp[0]
            else:
                wrap = cnt == (s + 1)
                qx = jnp.where(wrap, px[0], px[s + 1])
                qy = jnp.where(wrap, py[0], py[s + 1])
                cq = jnp.where(wrap, cp[0], cp[s + 1])
            ge_p = cp[s] >= 0
            ge_q = cq >= 0
            keep = vs & ge_p
            cross = vs & (ge_p != ge_q)
            denom = jnp.where(cross, cp[s] - cq, 1.0)
            t = cp[s] / denom
            ix = px[s] + t * (qx - px[s])
            iy = py[s] + t * (qy - py[s])
            for o in range(min(2 * s + 1, s_out)):
                c1 = keep & (off == o)
                nx[o] = jnp.where(c1, px[s], nx[o])
                ny[o] = jnp.where(c1, py[s], ny[o])
            off1 = off + keep.astype(jnp.int32)
            for o in range(min(2 * s + 2, s_out)):
                c2 = cross & (off1 == o)
                nx[o] = jnp.where(c2, ix, nx[o])
                ny[o] = jnp.where(c2, iy, ny[o])
            off = off1 + cross.astype(jnp.int32)
        px, py, cnt = nx, ny, off
    acc = jnp.zeros((TI, TJ), jnp.float32)
    for s in range(8):
        vs = cnt > s
        if s == 7:
            qx, qy = px[0], py[0]
        else:
            wrap = cnt == (s + 1)
            qx = jnp.where(wrap, px[0], px[s + 1])
            qy = jnp.where(wrap, py[0], py[s + 1])
        acc = acc + jnp.where(vs, px[s] * qy - qx * py[s], 0.0)
    inter = jnp.abs(acc) * 0.5
    union = area_i + area_j - inter
    ov = (union > 0) & (inter > IOU_THR * union)
    ov_ref[0] = ov.astype(jnp.float32)


def _nms_kernel(ov_ref, box_ref, sc_ref, sel_ref, cnt_ref):
    nb = sc_ref.shape[0]
    srow = sc_ref[...]
    lane = jax.lax.broadcasted_iota(jnp.int32, (nb, 1, PREP), 2)
    supp0 = (srow <= SCORE_THR).astype(jnp.float32)

    def body(i, supp):
        onehot = (lane == i).astype(jnp.float32)
        si = jnp.sum(supp * onehot, axis=(1, 2), keepdims=True)
        rowi = ov_ref[:, pl.ds(i, 1), :]
        cand = jnp.maximum(supp, jnp.where(lane > i, rowi, 0.0))
        return jnp.where(si == 0.0, cand, supp)

    supp = jax.lax.fori_loop(0, PRE, body, supp0)
    kept3 = 1.0 - supp
    li = jax.lax.broadcasted_iota(jnp.int32, (PREP, PREP), 0)
    lj = jax.lax.broadcasted_iota(jnp.int32, (PREP, PREP), 1)
    tri = (li <= lj).astype(jnp.float32)
    rows = jax.lax.broadcasted_iota(jnp.int32, (128, PREP), 0)
    for b in range(nb):
        kept = kept3[b]
        pos_incl = jnp.dot(kept, tri, preferred_element_type=jnp.float32)
        pos = (pos_incl - kept).astype(jnp.int32)
        sel_m = ((rows == jnp.broadcast_to(pos, (128, PREP)))
                 & (jnp.broadcast_to(kept, (128, PREP)) > 0)).astype(jnp.float32)
        sel_ref[b] = jnp.dot(sel_m, box_ref[b], preferred_element_type=jnp.float32)
        k = jnp.minimum(jnp.sum(kept), float(POST))
        cnt_ref[b] = jnp.reshape(k.astype(jnp.int32), (1, 1))


def kernel(anchors, box_preds, cls_preds, dir_cls_preds):
    B, N = cls_preds.shape[0], cls_preds.shape[1]
    NP = 20480
    logits = cls_preds[..., 0]
    lp = jnp.pad(logits, ((0, 0), (0, NP - N)), constant_values=-1e9)
    masked = pl.pallas_call(
        _mask_kernel,
        out_shape=jax.ShapeDtypeStruct((B * NP // 128, 128), jnp.float32),
        interpret=_INTERPRET,
    )(lp.reshape(B * NP // 128, 128))
    masked = masked.reshape(B, NP)
    top_vals, top_idx = jax.lax.top_k(masked, PRE)
    cand = jnp.concatenate([box_preds, anchors, dir_cls_preds], axis=-1)
    g = jnp.take_along_axis(cand, top_idx[..., None], axis=1)
    g = jnp.pad(g, ((0, 0), (0, PREP - PRE), (0, 0)))
    sc = jnp.pad(top_vals, ((0, 0), (0, PREP - PRE)), constant_values=-jnp.inf)

    outbox, cornT = pl.pallas_call(
        _decode_kernel,
        grid=(B,),
        in_specs=[pl.BlockSpec((1, PREP, 16), lambda b: (b, 0, 0)),
                  pl.BlockSpec((1, PREP, 1), lambda b: (b, 0, 0))],
        out_specs=[pl.BlockSpec((1, PREP, 8), lambda b: (b, 0, 0)),
                   pl.BlockSpec((1, PREP, 16), lambda b: (b, 0, 0))],
        out_shape=[jax.ShapeDtypeStruct((B, PREP, 8), jnp.float32),
                   jax.ShapeDtypeStruct((B, PREP, 16), jnp.float32)],
        interpret=_INTERPRET,
    )(g, sc[..., None])
    corn_by = jnp.swapaxes(cornT, 1, 2)

    ovm = pl.pallas_call(
        _overlap_kernel,
        grid=(B, _N_PAIR_TILES),
        in_specs=[pl.BlockSpec((1, TI, 16), lambda b, t: (b, _pair_tile_ij(t)[0], 0)),
                  pl.BlockSpec((1, 16, TJ), lambda b, t: (b, 0, _pair_tile_ij(t)[1])),],
        out_specs=pl.BlockSpec((1, TI, TJ),
                               lambda b, t: (b,) + _pair_tile_ij(t)),
        out_shape=jax.ShapeDtypeStruct((B, PREP, PREP), jnp.float32),
        interpret=_INTERPRET,
    )(cornT, corn_by)

    sel, cnt = pl.pallas_call(
        _nms_kernel,
        out_shape=[jax.ShapeDtypeStruct((B, 128, 8), jnp.float32),
                   jax.ShapeDtypeStruct((B, 1, 1), jnp.int32)],
        interpret=_INTERPRET,
    )(ovm, outbox, sc[:, None, :])

    boxes = sel[:, :POST, :7]
    scores = sel[:, :POST, 7]
    labels = jnp.zeros((B, POST), jnp.int32)
    counts = cnt.reshape(B)
    return boxes, scores, labels, counts


# speedup vs baseline: 1307.2842x; 1.2647x over previous
import math

import jax
import jax.numpy as jnp
from jax.experimental import pallas as pl
from jax.experimental.pallas import tpu as pltpu

_INTERPRET = False

SCORE_THR = 0.3
IOU_THR = 0.01
PRE = 1000
PREP = 1024
POST = 100
TWO_PI = 2.0 * math.pi
TI = 8
TJ = 256
_R = TJ // TI

_N_PAIR_TILES = (_R * (PREP // TJ) * (PREP // TJ + 1)) // 2


def _pair_tile_ij(t):
    tf = (1.0 + (8 * t).astype(jnp.float32) / _R).astype(jnp.float32)
    j = jnp.floor((jnp.sqrt(tf) - 1.0) * 0.5).astype(jnp.int32)
    i = t - (_R * j * (j + 1)) // 2
    return i, j


def _mask_kernel(x_ref, o_ref):
    s = jax.nn.sigmoid(x_ref[...])
    o_ref[...] = jnp.where(s > SCORE_THR, s, -jnp.inf)


def _decode_kernel(cand_ref, sc_ref, outbox_ref, corn_ref):
    c = cand_ref[0]
    sc = sc_ref[0]
    col = lambda a, i: jax.lax.slice_in_dim(a, i, i + 1, axis=1)
    xt, yt, zt, wt, lt, ht, rt = (col(c, i) for i in range(7))
    xa, ya, za, wa, la, ha, ra = (col(c, 7 + i) for i in range(7))
    d0, d1 = col(c, 14), col(c, 15)
    za = za + ha * 0.5
    diag = jnp.sqrt(la * la + wa * wa)
    xg = xt * diag + xa
    yg = yt * diag + ya
    zg = zt * ha + za
    lg = jnp.exp(lt) * la
    wg = jnp.exp(wt) * wa
    hg = jnp.exp(ht) * ha
    rg = rt + ra
    zg = zg - hg * 0.5
    dirlab = (d1 > d0).astype(jnp.float32)
    ang = rg - jnp.floor(rg / TWO_PI + 1.0) * TWO_PI + TWO_PI * dirlab
    score = jnp.where(sc > SCORE_THR, sc, 0.0)
    outbox_ref[0] = jnp.concatenate([xg, yg, zg, wg, lg, hg, ang, score], axis=1)
    ca, sa = jnp.cos(rg), jnp.sin(rg)
    hw, hl = wg * 0.5, lg * 0.5
    dxs = (-hw, hw, hw, -hw)
    dys = (-hl, -hl, hl, hl)
    cols = [ca * dxs[k] - sa * dys[k] + xg for k in range(4)]
    cols += [sa * dxs[k] + ca * dys[k] + yg for k in range(4)]
    cols.append(jnp.abs(wg * lg))
    z = jnp.zeros_like(xg)
    cols += [z] * 7
    corn_ref[0] = jnp.concatenate(cols, axis=1)


def _overlap_kernel(ci_ref, cj_ref, ov_ref):
    ci = ci_ref[0]
    cj = cj_ref[0]
    colT = lambda a, i: jax.lax.slice_in_dim(a, i, i + 1, axis=1)
    row = lambda a, i: jax.lax.slice_in_dim(a, i, i + 1, axis=0)
    px = [jnp.broadcast_to(colT(ci, k), (TI, TJ)) for k in range(4)]
    py = [jnp.broadcast_to(colT(ci, 4 + k), (TI, TJ)) for k in range(4)]
    area_i = colT(ci, 8)
    cxj = [row(cj, k) for k in range(4)]
    cyj = [row(cj, 4 + k) for k in range(4)]
    area_j = row(cj, 8)
    cnt = jnp.full((TI, TJ), 4, jnp.int32)
    for e in range(4):
        ax, ay = cxj[e], cyj[e]
        ex = cxj[(e + 1) % 4] - ax
        ey = cyj[(e + 1) % 4] - ay
        s_in = 4 + e
        s_out = 5 + e
        cp = [ex * (py[s] - ay) - ey * (px[s] - ax) for s in range(s_in)]
        nx = [jnp.zeros((TI, TJ), jnp.float32) for _ in range(s_out)]
        ny = [jnp.zeros((TI, TJ), jnp.float32) for _ in range(s_out)]
        off = jnp.zeros((TI, TJ), jnp.int32)
        for s in range(s_in):
            if e == 0:
                nid = (s + 1) % 4
                qx, qy, cq = px[nid], py[nid], cp[nid]
            elif s == s_in - 1:
                qx, qy, cq = px[0], py[0], cp[0]
            else:
                wrap = cnt == (s + 1)
                qx = jnp.where(wrap, px[0], px[s + 1])
                qy = jnp.where(wrap, py[0], py[s + 1])
                cq = jnp.where(wrap, cp[0], cp[s + 1])
            ge_p = cp[s] >= 0
            ge_q = cq >= 0
            if e == 0:
                keep = ge_p
                cross = ge_p != ge_q
            else:
                vs = cnt > s
                keep = vs & ge_p
                cross = vs & (ge_p != ge_q)
            denom = jnp.where(cross, cp[s] - cq, 1.0)
            t = cp[s] / denom
            ix = px[s] + t * (qx - px[s])
            iy = py[s] + t * (qy - py[s])
            for o in range(min(2 * s + 1, s_out)):
                c1 = keep & (off == o)
                nx[o] = jnp.where(c1, px[s], nx[o])
                ny[o] = jnp.where(c1, py[s], ny[o])
            off1 = off + keep.astype(jnp.int32)
            for o in range(min(2 * s + 2, s_out)):
                c2 = cross & (off1 == o)
                nx[o] = jnp.where(c2, ix, nx[o])
                ny[o] = jnp.where(c2, iy, ny[o])
            off = off1 + cross.astype(jnp.int32)
        px, py, cnt = nx, ny, off
    acc = jnp.zeros((TI, TJ), jnp.float32)
    for s in range(8):
        vs = cnt > s
        if s == 7:
            qx, qy = px[0], py[0]
        else:
            wrap = cnt == (s + 1)
            qx = jnp.where(wrap, px[0], px[s + 1])
            qy = jnp.where(wrap, py[0], py[s + 1])
        acc = acc + jnp.where(vs, px[s] * qy - qx * py[s], 0.0)
    inter = jnp.abs(acc) * 0.5
    union = area_i + area_j - inter
    ov = (union > 0) & (inter > IOU_THR * union)
    ov_ref[0] = ov.astype(jnp.float32)


def _nms_kernel(ov_ref, box_ref, sc_ref, sel_ref, cnt_ref):
    nb = sc_ref.shape[0]
    srow = sc_ref[...]
    lane = jax.lax.broadcasted_iota(jnp.int32, (nb, 1, PREP), 2)
    supp0 = (srow <= SCORE_THR).astype(jnp.float32)

    def body(i, supp):
        onehot = (lane == i).astype(jnp.float32)
        si = jnp.sum(supp * onehot, axis=(1, 2), keepdims=True)
        rowi = ov_ref[:, pl.ds(i, 1), :]
        cand = jnp.maximum(supp, jnp.where(lane > i, rowi, 0.0))
        return jnp.where(si == 0.0, cand, supp)

    supp = jax.lax.fori_loop(0, PRE, body, supp0)
    kept3 = 1.0 - supp
    li = jax.lax.broadcasted_iota(jnp.int32, (PREP, PREP), 0)
    lj = jax.lax.broadcasted_iota(jnp.int32, (PREP, PREP), 1)
    tri = (li <= lj).astype(jnp.float32)
    rows = jax.lax.broadcasted_iota(jnp.int32, (128, PREP), 0)
    for b in range(nb):
        kept = kept3[b]
        pos_incl = jnp.dot(kept, tri, preferred_element_type=jnp.float32)
        pos = (pos_incl - kept).astype(jnp.int32)
        sel_m = ((rows == jnp.broadcast_to(pos, (128, PREP)))
                 & (jnp.broadcast_to(kept, (128, PREP)) > 0)).astype(jnp.float32)
        sel_ref[b] = jnp.dot(sel_m, box_ref[b], preferred_element_type=jnp.float32)
        k = jnp.minimum(jnp.sum(kept), float(POST))
        cnt_ref[b] = jnp.reshape(k.astype(jnp.int32), (1, 1))


def kernel(anchors, box_preds, cls_preds, dir_cls_preds):
    B, N = cls_preds.shape[0], cls_preds.shape[1]
    NP = 20480
    logits = cls_preds[..., 0]
    lp = jnp.pad(logits, ((0, 0), (0, NP - N)), constant_values=-1e9)
    masked = pl.pallas_call(
        _mask_kernel,
        out_shape=jax.ShapeDtypeStruct((B * NP // 128, 128), jnp.float32),
        interpret=_INTERPRET,
    )(lp.reshape(B * NP // 128, 128))
    masked = masked.reshape(B, NP)
    top_vals, top_idx = jax.lax.top_k(masked, PRE)
    cand = jnp.concatenate([box_preds, anchors, dir_cls_preds], axis=-1)
    g = jnp.take_along_axis(cand, top_idx[..., None], axis=1)
    g = jnp.pad(g, ((0, 0), (0, PREP - PRE), (0, 0)))
    sc = jnp.pad(top_vals, ((0, 0), (0, PREP - PRE)), constant_values=-jnp.inf)

    outbox, cornT = pl.pallas_call(
        _decode_kernel,
        grid=(B,),
        in_specs=[pl.BlockSpec((1, PREP, 16), lambda b: (b, 0, 0)),
                  pl.BlockSpec((1, PREP, 1), lambda b: (b, 0, 0))],
        out_specs=[pl.BlockSpec((1, PREP, 8), lambda b: (b, 0, 0)),
                   pl.BlockSpec((1, PREP, 16), lambda b: (b, 0, 0))],
        out_shape=[jax.ShapeDtypeStruct((B, PREP, 8), jnp.float32),
                   jax.ShapeDtypeStruct((B, PREP, 16), jnp.float32)],
        interpret=_INTERPRET,
    )(g, sc[..., None])
    corn_by = jnp.swapaxes(cornT, 1, 2)

    ovm = pl.pallas_call(
        _overlap_kernel,
        grid=(B, _N_PAIR_TILES),
        in_specs=[pl.BlockSpec((1, TI, 16), lambda b, t: (b, _pair_tile_ij(t)[0], 0)),
                  pl.BlockSpec((1, 16, TJ), lambda b, t: (b, 0, _pair_tile_ij(t)[1])),],
        out_specs=pl.BlockSpec((1, TI, TJ),
                               lambda b, t: (b,) + _pair_tile_ij(t)),
        out_shape=jax.ShapeDtypeStruct((B, PREP, PREP), jnp.float32),
        compiler_params=pltpu.CompilerParams(
            dimension_semantics=("parallel", "parallel")),
        interpret=_INTERPRET,
    )(cornT, corn_by)

    sel, cnt = pl.pallas_call(
        _nms_kernel,
        out_shape=[jax.ShapeDtypeStruct((B, 128, 8), jnp.float32),
                   jax.ShapeDtypeStruct((B, 1, 1), jnp.int32)],
        interpret=_INTERPRET,
    )(ovm, outbox, sc[:, None, :])

    boxes = sel[:, :POST, :7]
    scores = sel[:, :POST, 7]
    labels = jnp.zeros((B, POST), jnp.int32)
    counts = cnt.reshape(B)
    return boxes, scores, labels, counts


# speedup vs baseline: 1390.8952x; 1.0640x over previous
import math

import jax
import jax.numpy as jnp
from jax.experimental import pallas as pl
from jax.experimental.pallas import tpu as pltpu

_INTERPRET = False

SCORE_THR = 0.3
IOU_THR = 0.01
PRE = 1000
PREP = 1024
POST = 100
TWO_PI = 2.0 * math.pi
TI = 16
TJ = 256
_R = TJ // TI

_N_PAIR_TILES = (_R * (PREP // TJ) * (PREP // TJ + 1)) // 2


def _pair_tile_ij(t):
    tf = (1.0 + (8 * t).astype(jnp.float32) / _R).astype(jnp.float32)
    j = jnp.floor((jnp.sqrt(tf) - 1.0) * 0.5).astype(jnp.int32)
    i = t - (_R * j * (j + 1)) // 2
    return i, j


def _mask_kernel(x_ref, o_ref):
    s = jax.nn.sigmoid(x_ref[...])
    o_ref[...] = jnp.where(s > SCORE_THR, s, -jnp.inf)


def _decode_kernel(cand_ref, sc_ref, outbox_ref, corn_ref):
    c = cand_ref[0]
    sc = sc_ref[0]
    col = lambda a, i: jax.lax.slice_in_dim(a, i, i + 1, axis=1)
    xt, yt, zt, wt, lt, ht, rt = (col(c, i) for i in range(7))
    xa, ya, za, wa, la, ha, ra = (col(c, 7 + i) for i in range(7))
    d0, d1 = col(c, 14), col(c, 15)
    za = za + ha * 0.5
    diag = jnp.sqrt(la * la + wa * wa)
    xg = xt * diag + xa
    yg = yt * diag + ya
    zg = zt * ha + za
    lg = jnp.exp(lt) * la
    wg = jnp.exp(wt) * wa
    hg = jnp.exp(ht) * ha
    rg = rt + ra
    zg = zg - hg * 0.5
    dirlab = (d1 > d0).astype(jnp.float32)
    ang = rg - jnp.floor(rg / TWO_PI + 1.0) * TWO_PI + TWO_PI * dirlab
    score = jnp.where(sc > SCORE_THR, sc, 0.0)
    outbox_ref[0] = jnp.concatenate([xg, yg, zg, wg, lg, hg, ang, score], axis=1)
    ca, sa = jnp.cos(rg), jnp.sin(rg)
    hw, hl = wg * 0.5, lg * 0.5
    dxs = (-hw, hw, hw, -hw)
    dys = (-hl, -hl, hl, hl)
    cols = [ca * dxs[k] - sa * dys[k] + xg for k in range(4)]
    cols += [sa * dxs[k] + ca * dys[k] + yg for k in range(4)]
    cols.append(jnp.abs(wg * lg))
    z = jnp.zeros_like(xg)
    cols += [z] * 7
    corn_ref[0] = jnp.concatenate(cols, axis=1)


def _overlap_kernel(ci_ref, cj_ref, ov_ref):
    ci = ci_ref[0]
    cj = cj_ref[0]
    colT = lambda a, i: jax.lax.slice_in_dim(a, i, i + 1, axis=1)
    row = lambda a, i: jax.lax.slice_in_dim(a, i, i + 1, axis=0)
    px = [jnp.broadcast_to(colT(ci, k), (TI, TJ)) for k in range(4)]
    py = [jnp.broadcast_to(colT(ci, 4 + k), (TI, TJ)) for k in range(4)]
    area_i = colT(ci, 8)
    cxj = [row(cj, k) for k in range(4)]
    cyj = [row(cj, 4 + k) for k in range(4)]
    area_j = row(cj, 8)
    cnt = jnp.full((TI, TJ), 4, jnp.int32)
    for e in range(4):
        ax, ay = cxj[e], cyj[e]
        ex = cxj[(e + 1) % 4] - ax
        ey = cyj[(e + 1) % 4] - ay
        s_in = 4 + e
        s_out = 5 + e
        cp = [ex * (py[s] - ay) - ey * (px[s] - ax) for s in range(s_in)]
        nx = [jnp.zeros((TI, TJ), jnp.float32) for _ in range(s_out)]
        ny = [jnp.zeros((TI, TJ), jnp.float32) for _ in range(s_out)]
        off = jnp.zeros((TI, TJ), jnp.int32)
        for s in range(s_in):
            if e == 0:
                nid = (s + 1) % 4
                qx, qy, cq = px[nid], py[nid], cp[nid]
            elif s == s_in - 1:
                qx, qy, cq = px[0], py[0], cp[0]
            else:
                wrap = cnt == (s + 1)
                qx = jnp.where(wrap, px[0], px[s + 1])
                qy = jnp.where(wrap, py[0], py[s + 1])
                cq = jnp.where(wrap, cp[0], cp[s + 1])
            ge_p = cp[s] >= 0
            ge_q = cq >= 0
            if e == 0:
                keep = ge_p
                cross = ge_p != ge_q
            else:
                vs = cnt > s
                keep = vs & ge_p
                cross = vs & (ge_p != ge_q)
            denom = jnp.where(cross, cp[s] - cq, 1.0)
            t = cp[s] / denom
            ix = px[s] + t * (qx - px[s])
            iy = py[s] + t * (qy - py[s])
            for o in range(min(2 * s + 1, s_out)):
                c1 = keep & (off == o)
                nx[o] = jnp.where(c1, px[s], nx[o])
                ny[o] = jnp.where(c1, py[s], ny[o])
            off1 = off + keep.astype(jnp.int32)
            for o in range(min(2 * s + 2, s_out)):
                c2 = cross & (off1 == o)
                nx[o] = jnp.where(c2, ix, nx[o])
                ny[o] = jnp.where(c2, iy, ny[o])
            off = off1 + cross.astype(jnp.int32)
        px, py, cnt = nx, ny, off
    acc = jnp.zeros((TI, TJ), jnp.float32)
    for s in range(8):
        vs = cnt > s
        if s == 7:
            qx, qy = px[0], py[0]
        else:
            wrap = cnt == (s + 1)
            qx = jnp.where(wrap, px[0], px[s + 1])
            qy = jnp.where(wrap, py[0], py[s + 1])
        acc = acc + jnp.where(vs, px[s] * qy - qx * py[s], 0.0)
    inter = jnp.abs(acc) * 0.5
    union = area_i + area_j - inter
    ov = (union > 0) & (inter > IOU_THR * union)
    ov_ref[0] = ov.astype(jnp.float32)


def _nms_kernel(ov_ref, box_ref, sc_ref, sel_ref, cnt_ref):
    nb = sc_ref.shape[0]
    srow = sc_ref[...]
    lane = jax.lax.broadcasted_iota(jnp.int32, (nb, 1, PREP), 2)
    supp0 = (srow <= SCORE_THR).astype(jnp.float32)

    def body(i, supp):
        onehot = (lane == i).astype(jnp.float32)
        si = jnp.sum(supp * onehot, axis=(1, 2), keepdims=True)
        rowi = ov_ref[:, pl.ds(i, 1), :]
        cand = jnp.maximum(supp, jnp.where(lane > i, rowi, 0.0))
        return jnp.where(si == 0.0, cand, supp)

    supp = jax.lax.fori_loop(0, PRE, body, supp0)
    kept3 = 1.0 - supp
    li = jax.lax.broadcasted_iota(jnp.int32, (PREP, PREP), 0)
    lj = jax.lax.broadcasted_iota(jnp.int32, (PREP, PREP), 1)
    tri = (li <= lj).astype(jnp.float32)
    rows = jax.lax.broadcasted_iota(jnp.int32, (128, PREP), 0)
    for b in range(nb):
        kept = kept3[b]
        pos_incl = jnp.dot(kept, tri, preferred_element_type=jnp.float32)
        pos = (pos_incl - kept).astype(jnp.int32)
        sel_m = ((rows == jnp.broadcast_to(pos, (128, PREP)))
                 & (jnp.broadcast_to(kept, (128, PREP)) > 0)).astype(jnp.float32)
        sel_ref[b] = jnp.dot(sel_m, box_ref[b], preferred_element_type=jnp.float32)
        k = jnp.minimum(jnp.sum(kept), float(POST))
        cnt_ref[b] = jnp.reshape(k.astype(jnp.int32), (1, 1))


def kernel(anchors, box_preds, cls_preds, dir_cls_preds):
    B, N = cls_preds.shape[0], cls_preds.shape[1]
    NP = 20480
    logits = cls_preds[..., 0]
    lp = jnp.pad(logits, ((0, 0), (0, NP - N)), constant_values=-1e9)
    masked = pl.pallas_call(
        _mask_kernel,
        out_shape=jax.ShapeDtypeStruct((B * NP // 128, 128), jnp.float32),
        interpret=_INTERPRET,
    )(lp.reshape(B * NP // 128, 128))
    masked = masked.reshape(B, NP)
    top_vals, top_idx = jax.lax.top_k(masked, PRE)
    cand = jnp.concatenate([box_preds, anchors, dir_cls_preds], axis=-1)
    g = jnp.take_along_axis(cand, top_idx[..., None], axis=1)
    g = jnp.pad(g, ((0, 0), (0, PREP - PRE), (0, 0)))
    sc = jnp.pad(top_vals, ((0, 0), (0, PREP - PRE)), constant_values=-jnp.inf)

    outbox, cornT = pl.pallas_call(
        _decode_kernel,
        grid=(B,),
        in_specs=[pl.BlockSpec((1, PREP, 16), lambda b: (b, 0, 0)),
                  pl.BlockSpec((1, PREP, 1), lambda b: (b, 0, 0))],
        out_specs=[pl.BlockSpec((1, PREP, 8), lambda b: (b, 0, 0)),
                   pl.BlockSpec((1, PREP, 16), lambda b: (b, 0, 0))],
        out_shape=[jax.ShapeDtypeStruct((B, PREP, 8), jnp.float32),
                   jax.ShapeDtypeStruct((B, PREP, 16), jnp.float32)],
        interpret=_INTERPRET,
    )(g, sc[..., None])
    corn_by = jnp.swapaxes(cornT, 1, 2)

    ovm = pl.pallas_call(
        _overlap_kernel,
        grid=(B, _N_PAIR_TILES),
        in_specs=[pl.BlockSpec((1, TI, 16), lambda b, t: (b, _pair_tile_ij(t)[0], 0)),
                  pl.BlockSpec((1, 16, TJ), lambda b, t: (b, 0, _pair_tile_ij(t)[1])),],
        out_specs=pl.BlockSpec((1, TI, TJ),
                               lambda b, t: (b,) + _pair_tile_ij(t)),
        out_shape=jax.ShapeDtypeStruct((B, PREP, PREP), jnp.float32),
        compiler_params=pltpu.CompilerParams(
            dimension_semantics=("parallel", "parallel")),
        interpret=_INTERPRET,
    )(cornT, corn_by)

    sel, cnt = pl.pallas_call(
        _nms_kernel,
        out_shape=[jax.ShapeDtypeStruct((B, 128, 8), jnp.float32),
                   jax.ShapeDtypeStruct((B, 1, 1), jnp.int32)],
        interpret=_INTERPRET,
    )(ovm, outbox, sc[:, None, :])

    boxes = sel[:, :POST, :7]
    scores = sel[:, :POST, 7]
    labels = jnp.zeros((B, POST), jnp.int32)
    counts = cnt.reshape(B)
    return boxes, scores, labels, counts
